# baseline (speedup 1.0000x reference)
import functools

import jax
import jax.numpy as jnp
from jax import lax
from jax.experimental import pallas as pl
from jax.experimental.pallas import tpu as pltpu
from jax.experimental.pallas import tpu_sc as plsc


def _make_gather(V, D, B, H):
    info = plsc.get_sparse_core_info()
    NC, NS, L = info.num_cores, info.num_subcores, info.num_lanes
    NW = NC * NS
    assert B % (NW * 128) == 0
    rows_per_w = B // NW
    DT = D // 8
    BT = B // 128
    bt_per_w = rows_per_w // 128
    NBUF = 3
    n_outer = -(-H // NBUF)

    mesh = plsc.VectorSubcoreMesh(core_axis_name="c", subcore_axis_name="s")

    @functools.partial(
        pl.kernel,
        mesh=mesh,
        compiler_params=pltpu.CompilerParams(
            use_tc_tiling_on_sc=False, needs_layout_passes=False
        ),
        out_type=jax.ShapeDtypeStruct((H, DT, BT, 8, 128), jnp.float32),
        scratch_types=[
            pltpu.VMEM((H, rows_per_w), jnp.int32),
            *[pltpu.VMEM((rows_per_w, D), jnp.float32) for _ in range(NBUF)],
            *[pltpu.VMEM((DT, bt_per_w, 8, 128), jnp.float32) for _ in range(NBUF)],
            *[pltpu.SemaphoreType.DMA for _ in range(2 * NBUF)],
        ],
    )
    def gather_kernel(table_hbm, xt_hbm, out_hbm, idx_t, *rest):
        rows = rest[:NBUF]
        tbuf = rest[NBUF : 2 * NBUF]
        gsem = rest[2 * NBUF : 3 * NBUF]
        wsem = rest[3 * NBUF :]
        wid = lax.axis_index("s") * NC + lax.axis_index("c")
        base_row = wid * rows_per_w
        bt0 = wid * bt_per_w

        pltpu.sync_copy(xt_hbm.at[:, pl.ds(base_row, rows_per_w)], idx_t)

        def start_gather(h, b):
            pltpu.async_copy(table_hbm.at[idx_t.at[h]], rows[b], gsem[b])

        def wait_gather(b):
            pltpu.make_async_copy(
                table_hbm.at[pl.ds(0, rows_per_w)], rows[b], gsem[b]
            ).wait()

        def transpose_slab(b):
            for bt in range(bt_per_w):
                for jg in range(128 // L):
                    bvec = lax.iota(jnp.int32, L) + (bt * 128 + jg * L)
                    vs = [
                        plsc.load_gather(
                            rows[b], [bvec, jnp.full((L,), d, jnp.int32)]
                        )
                        for d in range(D)
                    ]
                    for d in range(D):
                        tbuf[b][d // 8, bt, d % 8, pl.ds(jg * L, L)] = vs[d]

        def start_write(h, b):
            pltpu.async_copy(
                tbuf[b], out_hbm.at[h, :, pl.ds(bt0, bt_per_w)], wsem[b]
            )

        def wait_write(b):
            pltpu.make_async_copy(
                tbuf[b], out_hbm.at[0, :, pl.ds(0, bt_per_w)], wsem[b]
            ).wait()

        for b in range(NBUF):
            start_gather(b, b)

        def body(g, carry):
            for b in range(NBUF):
                h = g * NBUF + b

                @pl.when(h < H)
                def _():
                    wait_gather(b)

                    @pl.when(g > 0)
                    def _():
                        wait_write(b)

                    transpose_slab(b)
                    start_write(h, b)

                    @pl.when(h + NBUF < H)
                    def _():
                        start_gather(h + NBUF, b)

            return carry

        lax.fori_loop(0, n_outer, body, 0)

        for b in range(NBUF):
            wait_write(b)

    return gather_kernel


def kernel(table, x):
    V, D = table.shape
    B, H = x.shape
    out5 = _make_gather(V, D, B, H)(table, x.T)
    return out5.transpose(2, 4, 0, 1, 3).reshape(B, H, D)


# speedup vs baseline: 1.0056x; 1.0056x over previous
import functools

import jax
import jax.numpy as jnp
from jax import lax
from jax.experimental import pallas as pl
from jax.experimental.pallas import tpu as pltpu
from jax.experimental.pallas import tpu_sc as plsc


def _make_gather(V, D, B, H):
    info = plsc.get_sparse_core_info()
    NC, NS, L = info.num_cores, info.num_subcores, info.num_lanes
    NW = NC * NS
    assert B % (NW * 128) == 0
    rows_per_w = B // NW
    DT = D // 8
    BT = B // 128
    bt_per_w = rows_per_w // 128
    NBUF = 2
    assert H % NBUF == 0

    mesh = plsc.VectorSubcoreMesh(core_axis_name="c", subcore_axis_name="s")

    @functools.partial(
        pl.kernel,
        mesh=mesh,
        compiler_params=pltpu.CompilerParams(
            use_tc_tiling_on_sc=False, needs_layout_passes=False
        ),
        out_type=jax.ShapeDtypeStruct((H, DT, BT, 8, 128), jnp.float32),
        scratch_types=[
            pltpu.VMEM((H, rows_per_w), jnp.int32),
            *[pltpu.VMEM((rows_per_w, D), jnp.float32) for _ in range(NBUF)],
            *[pltpu.VMEM((DT, bt_per_w, 8, 128), jnp.float32) for _ in range(NBUF)],
            *[pltpu.SemaphoreType.DMA for _ in range(2 * NBUF)],
        ],
    )
    def gather_kernel(table_hbm, xt_hbm, out_hbm, idx_t, *rest):
        rows = rest[:NBUF]
        tbuf = rest[NBUF : 2 * NBUF]
        gsem = rest[2 * NBUF : 3 * NBUF]
        wsem = rest[3 * NBUF :]
        wid = lax.axis_index("s") * NC + lax.axis_index("c")
        base_row = wid * rows_per_w
        bt0 = wid * bt_per_w

        pltpu.sync_copy(xt_hbm.at[:, pl.ds(base_row, rows_per_w)], idx_t)

        def start_gather(h, b):
            pltpu.async_copy(table_hbm.at[idx_t.at[h]], rows[b], gsem[b])

        def wait_gather(b):
            pltpu.make_async_copy(
                table_hbm.at[pl.ds(0, rows_per_w)], rows[b], gsem[b]
            ).wait()

        def transpose_slab(b):
            for bt in range(bt_per_w):
                for jg in range(128 // L):
                    bvec = lax.iota(jnp.int32, L) + (bt * 128 + jg * L)
                    vs = [
                        plsc.load_gather(
                            rows[b], [bvec, jnp.full((L,), d, jnp.int32)]
                        )
                        for d in range(D)
                    ]
                    for d in range(D):
                        tbuf[b][d // 8, bt, d % 8, pl.ds(jg * L, L)] = vs[d]

        def start_write(h, b):
            pltpu.async_copy(
                tbuf[b], out_hbm.at[h, :, pl.ds(bt0, bt_per_w)], wsem[b]
            )

        def wait_write(b):
            pltpu.make_async_copy(
                tbuf[b], out_hbm.at[0, :, pl.ds(0, bt_per_w)], wsem[b]
            ).wait()

        for b in range(NBUF):
            start_gather(b, b)

        def body(g, carry):
            for b in range(NBUF):
                h = g * NBUF + b
                wait_gather(b)

                @pl.when(g > 0)
                def _():
                    wait_write(b)

                transpose_slab(b)
                start_write(h, b)

                @pl.when(g < H // NBUF - 1)
                def _():
                    start_gather(h + NBUF, b)

            return carry

        lax.fori_loop(0, H // NBUF, body, 0)

        for b in range(NBUF):
            wait_write(b)

    return gather_kernel


def kernel(table, x):
    V, D = table.shape
    B, H = x.shape
    out5 = _make_gather(V, D, B, H)(table, x.T)
    return out5.transpose(2, 4, 0, 1, 3).reshape(B, H, D)
